# baseline (speedup 1.0000x reference)
import functools

import jax
import jax.numpy as jnp
from jax import lax
from jax.experimental import pallas as pl
from jax.experimental.pallas import tpu as pltpu
from jax.experimental.pallas import tpu_sc as plsc

N = 8 * 512 * 512
NW = 32
ROWS = N // 128
ROWS_W = ROWS // NW
CH_ROWS = 128
NCHUNKS = ROWS_W // CH_ROWS
NB = 16384
HROWS = 4 * NB // 128

_mesh = plsc.VectorSubcoreMesh(core_axis_name="c", subcore_axis_name="s")


@functools.partial(
    pl.kernel,
    mesh=_mesh,
    compiler_params=pltpu.CompilerParams(needs_layout_passes=False),
    out_type=jax.ShapeDtypeStruct((NW * HROWS, 128), jnp.float32),
    scratch_types=[
        pltpu.VMEM((HROWS, 128), jnp.float32),
        pltpu.VMEM((CH_ROWS, 128), jnp.float32),
        pltpu.VMEM((CH_ROWS, 128), jnp.float32),
        pltpu.SemaphoreType.DMA((2,)),
    ],
)
def _sc_hist(v_hbm, out_hbm, hist_v, buf_a, buf_b, sem):
    wid = lax.axis_index("s") * 2 + lax.axis_index("c")
    base = wid * ROWS_W
    bufs = (buf_a, buf_b)

    @plsc.parallel_loop(0, HROWS * 8, unroll=8)
    def _(i):
        hist_v[i >> 3, pl.ds((i & 7) * 16, 16)] = jnp.zeros((16,), jnp.float32)

    ones16 = jnp.ones((16,), jnp.float32)

    def start(ci):
        pltpu.async_copy(v_hbm.at[pl.ds(base + ci * CH_ROWS, CH_ROWS)],
                         bufs[ci % 2], sem.at[ci % 2])

    def wait(ci):
        pltpu.make_async_copy(v_hbm.at[pl.ds(base + ci * CH_ROWS, CH_ROWS)],
                              bufs[ci % 2], sem.at[ci % 2]).wait()

    start(0)
    for ci in range(NCHUNKS):
        if ci + 1 < NCHUNKS:
            start(ci + 1)
        wait(ci)
        buf = bufs[ci % 2]

        @plsc.parallel_loop(0, CH_ROWS, unroll=2)
        def _(j):
            for k in range(8):
                v = buf[j, pl.ds(k * 16, 16)]
                bits = plsc.bitcast(v, jnp.int32)
                lab = bits & 1
                bid = lax.shift_right_logical(bits, 17)
                idx = jnp.where(v > 0.0, bid, 0) + lab * NB
                row = lax.shift_right_logical(idx, 7)
                col = idx & 127
                r = jnp.maximum(v, 0.0)
                plsc.addupdate_scatter(hist_v, [row, col], ones16)
                plsc.addupdate_scatter(hist_v, [row + 2 * NB // 128, col], r)

    pltpu.sync_copy(hist_v, out_hbm.at[pl.ds(wid * HROWS, HROWS)])


def _tc_scan_kernel(h_ref, out_ref):
    X = jnp.sum(h_ref[...].reshape(NW, 4, 128, 128), axis=0)
    n0 = X[0]
    n1 = X[1]
    s0 = X[2]
    s1 = X[3]
    row = lax.broadcasted_iota(jnp.int32, (128, 128), 0)
    col = lax.broadcasted_iota(jnp.int32, (128, 128), 1)
    tril = (row >= col).astype(jnp.float32)
    triu1 = (row < col).astype(jnp.float32)
    F0 = jnp.dot(n0, tril, preferred_element_type=jnp.float32)
    F1 = jnp.dot(n1, tril, preferred_element_type=jnp.float32)
    t0 = F0[:, 0:1]
    t1 = F1[:, 0:1]
    E0 = jnp.dot(triu1, t0, preferred_element_type=jnp.float32)
    E1 = jnp.dot(triu1, t1, preferred_element_type=jnp.float32)
    f_incl = F0 + E0
    c_e = F1 + E1
    f_s = f_incl - n0
    p = jnp.sum(n1)
    d_s = jnp.maximum(p + f_s, 1.0)
    d_e = jnp.maximum(p + f_s + n0, 1.0)
    pos = s1 / d_s
    neg = s0 * (p - c_e) / (d_s * d_e)
    out_ref[...] = jnp.sum(pos + neg)[None, None]


def _tc_pack_kernel(lgt_ref, tgt_ref, out_ref):
    x = lgt_ref[0]
    t = tgt_ref[0]
    e = 1.0 - x * (2 * t - 1).astype(jnp.float32)
    bits = lax.bitcast_convert_type(e, jnp.int32)
    out_ref[...] = lax.bitcast_convert_type((bits & -2) | t, jnp.float32)


def kernel(logits, target):
    tgt = target.astype(jnp.int32)
    v2d = pl.pallas_call(
        _tc_pack_kernel,
        grid=(8, 4),
        in_specs=[
            pl.BlockSpec((1, 512, 128), lambda b, c: (b, 0, c)),
            pl.BlockSpec((1, 512, 128), lambda b, c: (b, 0, c)),
        ],
        out_specs=pl.BlockSpec((512, 128), lambda b, c: (b * 4 + c, 0)),
        out_shape=jax.ShapeDtypeStruct((ROWS, 128), jnp.float32),
    )(logits, tgt)
    hists = _sc_hist(v2d)
    loss = pl.pallas_call(
        _tc_scan_kernel,
        out_shape=jax.ShapeDtypeStruct((1, 1), jnp.float32),
    )(hists)
    return loss[0, 0]


# speedup vs baseline: 1.2223x; 1.2223x over previous
import functools

import jax
import jax.numpy as jnp
from jax import lax
from jax.experimental import pallas as pl
from jax.experimental.pallas import tpu as pltpu
from jax.experimental.pallas import tpu_sc as plsc

N = 8 * 512 * 512
NW = 32
ROWS = N // 128
ROWS_W = ROWS // NW
CH_ROWS = 128
NCHUNKS = ROWS_W // CH_ROWS
NB = 16384
HROWS = 4 * NB // 128

_mesh = plsc.VectorSubcoreMesh(core_axis_name="c", subcore_axis_name="s")


@functools.partial(
    pl.kernel,
    mesh=_mesh,
    compiler_params=pltpu.CompilerParams(needs_layout_passes=False),
    out_type=jax.ShapeDtypeStruct((NW * HROWS, 128), jnp.float32),
    scratch_types=[
        pltpu.VMEM((HROWS, 128), jnp.float32),
        pltpu.VMEM((CH_ROWS, 128), jnp.float32),
        pltpu.VMEM((CH_ROWS, 128), jnp.float32),
        pltpu.SemaphoreType.DMA((2,)),
    ],
)
def _sc_hist(v_hbm, out_hbm, hist_v, buf_a, buf_b, sem):
    wid = lax.axis_index("s") * 2 + lax.axis_index("c")
    base = wid * ROWS_W
    bufs = (buf_a, buf_b)

    @plsc.parallel_loop(0, HROWS * 8, unroll=8)
    def _(i):
        hist_v[i >> 3, pl.ds((i & 7) * 16, 16)] = jnp.zeros((16,), jnp.float32)

    ones16 = jnp.ones((16,), jnp.float32)

    def start(ci):
        pltpu.async_copy(v_hbm.at[pl.ds(base + ci * CH_ROWS, CH_ROWS)],
                         bufs[ci % 2], sem.at[ci % 2])

    def wait(ci):
        pltpu.make_async_copy(v_hbm.at[pl.ds(base + ci * CH_ROWS, CH_ROWS)],
                              bufs[ci % 2], sem.at[ci % 2]).wait()

    start(0)
    for ci in range(NCHUNKS):
        if ci + 1 < NCHUNKS:
            start(ci + 1)
        wait(ci)
        buf = bufs[ci % 2]

        @plsc.parallel_loop(0, CH_ROWS * 8, unroll=8)
        def _(j):
            v = buf[j >> 3, pl.ds((j & 7) * 16, 16)]
            bits = plsc.bitcast(v, jnp.int32)
            lab = bits & 1
            bid = lax.shift_right_logical(bits, 17)
            idx = jnp.where(v > 0.0, bid, 0) + lab * NB
            row = lax.shift_right_logical(idx, 7)
            col = idx & 127
            r = jnp.maximum(v, 0.0)
            plsc.addupdate_scatter(hist_v, [row, col], ones16)
            plsc.addupdate_scatter(hist_v, [row + 2 * NB // 128, col], r)

    pltpu.sync_copy(hist_v, out_hbm.at[pl.ds(wid * HROWS, HROWS)])


def _tc_scan_kernel(h_ref, out_ref):
    X = jnp.sum(h_ref[...].reshape(NW, 4, 128, 128), axis=0)
    n0 = X[0]
    n1 = X[1]
    s0 = X[2]
    s1 = X[3]
    row = lax.broadcasted_iota(jnp.int32, (128, 128), 0)
    col = lax.broadcasted_iota(jnp.int32, (128, 128), 1)
    tril = (row >= col).astype(jnp.float32)
    triu1 = (row < col).astype(jnp.float32)
    F0 = jnp.dot(n0, tril, preferred_element_type=jnp.float32)
    F1 = jnp.dot(n1, tril, preferred_element_type=jnp.float32)
    t0 = F0[:, 0:1]
    t1 = F1[:, 0:1]
    E0 = jnp.dot(triu1, t0, preferred_element_type=jnp.float32)
    E1 = jnp.dot(triu1, t1, preferred_element_type=jnp.float32)
    f_incl = F0 + E0
    c_e = F1 + E1
    f_s = f_incl - n0
    p = jnp.sum(n1)
    d_s = jnp.maximum(p + f_s, 1.0)
    d_e = jnp.maximum(p + f_s + n0, 1.0)
    pos = s1 / d_s
    neg = s0 * (p - c_e) / (d_s * d_e)
    out_ref[...] = jnp.sum(pos + neg)[None, None]


def _tc_pack_kernel(lgt_ref, tgt_ref, out_ref):
    x = lgt_ref[0]
    t = tgt_ref[0]
    e = 1.0 - x * (2 * t - 1).astype(jnp.float32)
    bits = lax.bitcast_convert_type(e, jnp.int32)
    v = lax.bitcast_convert_type((bits & -2) | t, jnp.float32)
    out_ref[...] = jnp.concatenate([v[:, i * 128:(i + 1) * 128]
                                    for i in range(4)], axis=0)


def kernel(logits, target):
    tgt = target.astype(jnp.int32)
    v2d = pl.pallas_call(
        _tc_pack_kernel,
        grid=(8,),
        in_specs=[
            pl.BlockSpec((1, 512, 512), lambda b: (b, 0, 0)),
            pl.BlockSpec((1, 512, 512), lambda b: (b, 0, 0)),
        ],
        out_specs=pl.BlockSpec((2048, 128), lambda b: (b, 0)),
        out_shape=jax.ShapeDtypeStruct((ROWS, 128), jnp.float32),
    )(logits, tgt)
    hists = _sc_hist(v2d)
    loss = pl.pallas_call(
        _tc_scan_kernel,
        out_shape=jax.ShapeDtypeStruct((1, 1), jnp.float32),
    )(hists)
    return loss[0, 0]
